# Initial kernel scaffold; baseline (speedup 1.0000x reference)
import jax
import jax.numpy as jnp
from jax.experimental import pallas as pl


def kernel(global_features, speaker, Wq, Wk, Wv, rgcn_weight, rgcn_root, rgcn_bias, gcn_lin_rel, gcn_lin_root, gcn_bias):
    raise NotImplementedError("write your pallas kernel here")


# speedup vs baseline: 42.7570x; 42.7570x over previous
import jax
import jax.numpy as jnp
from jax.experimental import pallas as pl
from jax.experimental.pallas import tpu as pltpu

S = 8
H = 256
E = S * S


def _body(et_ref, gf_ref, wq_ref, wk_ref, w_ref, root_ref, rb_ref,
          lrel_ref, lroot_ref, gb_ref, out_ref, wgf_ref, acc_ref):
    e = pl.program_id(0)

    @pl.when(e == 0)
    def _init():
        gf = gf_ref[...]
        q = jnp.dot(gf, wq_ref[...], preferred_element_type=jnp.float32)
        k = jnp.dot(gf, wk_ref[...], preferred_element_type=jnp.float32)
        scores = jnp.dot(q, k.T, preferred_element_type=jnp.float32) * (1.0 / 16.0)
        scores = scores - jnp.max(scores, axis=-1, keepdims=True)
        ex = jnp.exp(scores)
        attn = ex / jnp.sum(ex, axis=-1, keepdims=True)
        msgs = attn[:, :, None] * gf[:, None, :]
        wgf_ref[...] = msgs.reshape(E, H)
        acc_ref[...] = jnp.dot(gf, root_ref[...],
                               preferred_element_type=jnp.float32) + rb_ref[...]

    d = jax.lax.rem(e, S)
    msg = jnp.dot(wgf_ref[pl.ds(e, 1), :], w_ref[0],
                  preferred_element_type=jnp.float32)
    acc_ref[pl.ds(d, 1), :] += msg

    @pl.when(e == E - 1)
    def _finish():
        x1 = acc_ref[...]
        agg = jnp.broadcast_to(jnp.sum(x1, axis=0, keepdims=True), (S, H))
        out2 = (jnp.dot(agg, lrel_ref[...], preferred_element_type=jnp.float32)
                + jnp.dot(x1, lroot_ref[...], preferred_element_type=jnp.float32)
                + gb_ref[...])
        out_ref[:, :H] = out2
        out_ref[:, H:] = gf_ref[...]


def kernel(global_features, speaker, Wq, Wk, Wv, rgcn_weight, rgcn_root,
           rgcn_bias, gcn_lin_rel, gcn_lin_root, gcn_bias):
    del Wv
    spk = speaker.astype(jnp.int32)
    src = jnp.repeat(jnp.arange(S, dtype=jnp.int32), S)
    dst = jnp.tile(jnp.arange(S, dtype=jnp.int32), S)
    et = (spk[src] * S + spk[dst]) * 2 + (src >= dst).astype(jnp.int32)

    full = lambda shape: pl.BlockSpec(shape, lambda e, et_ref: (0,) * len(shape))
    grid_spec = pltpu.PrefetchScalarGridSpec(
        num_scalar_prefetch=1,
        grid=(E,),
        in_specs=[
            full((S, H)),
            full((H, H)),
            full((H, H)),
            pl.BlockSpec((1, H, H), lambda e, et_ref: (et_ref[e], 0, 0)),
            full((H, H)),
            full((1, H)),
            full((H, H)),
            full((H, H)),
            full((1, H)),
        ],
        out_specs=pl.BlockSpec((S, 2 * H), lambda e, et_ref: (0, 0)),
        scratch_shapes=[
            pltpu.VMEM((E, H), jnp.float32),
            pltpu.VMEM((S, H), jnp.float32),
        ],
    )
    return pl.pallas_call(
        _body,
        grid_spec=grid_spec,
        out_shape=jax.ShapeDtypeStruct((S, 2 * H), jnp.float32),
    )(et, global_features, Wq, Wk, rgcn_weight, rgcn_root,
      rgcn_bias.reshape(1, H), gcn_lin_rel, gcn_lin_root,
      gcn_bias.reshape(1, H))


# speedup vs baseline: 108.7834x; 2.5442x over previous
import jax
import jax.numpy as jnp
from jax.experimental import pallas as pl
from jax.experimental.pallas import tpu as pltpu

S = 8
H = 256
E = S * S


def _body(et_ref, gf_ref, wq_ref, wk_ref, *rest):
    w_refs = rest[:S]
    (root_ref, rb_ref, lrel_ref, lroot_ref, gb_ref,
     out_ref, wgf_ref, acc_ref) = rest[S:]
    s = pl.program_id(0)

    @pl.when(s == 0)
    def _init():
        gf = gf_ref[...]
        q = jnp.dot(gf, wq_ref[...], preferred_element_type=jnp.float32)
        k = jnp.dot(gf, wk_ref[...], preferred_element_type=jnp.float32)
        scores = jnp.dot(q, k.T, preferred_element_type=jnp.float32) * (1.0 / 16.0)
        scores = scores - jnp.max(scores, axis=-1, keepdims=True)
        ex = jnp.exp(scores)
        attn = ex / jnp.sum(ex, axis=-1, keepdims=True)
        msgs = attn[:, :, None] * gf[:, None, :]
        wgf_ref[...] = msgs.reshape(E, H)
        acc_ref[...] = jnp.dot(gf, root_ref[...],
                               preferred_element_type=jnp.float32) + rb_ref[...]

    rows = wgf_ref[pl.ds(S * s, S), :]
    msgs = [jnp.dot(rows[j:j + 1, :], w_refs[j][0],
                    preferred_element_type=jnp.float32) for j in range(S)]
    acc_ref[...] += jnp.concatenate(msgs, axis=0)

    @pl.when(s == S - 1)
    def _finish():
        x1 = acc_ref[...]
        agg = jnp.broadcast_to(jnp.sum(x1, axis=0, keepdims=True), (S, H))
        out2 = (jnp.dot(agg, lrel_ref[...], preferred_element_type=jnp.float32)
                + jnp.dot(x1, lroot_ref[...], preferred_element_type=jnp.float32)
                + gb_ref[...])
        out_ref[:, :H] = out2
        out_ref[:, H:] = gf_ref[...]


def kernel(global_features, speaker, Wq, Wk, Wv, rgcn_weight, rgcn_root,
           rgcn_bias, gcn_lin_rel, gcn_lin_root, gcn_bias):
    del Wv
    spk = speaker.astype(jnp.int32)
    src = jnp.repeat(jnp.arange(S, dtype=jnp.int32), S)
    dst = jnp.tile(jnp.arange(S, dtype=jnp.int32), S)
    et = (spk[src] * S + spk[dst]) * 2 + (src >= dst).astype(jnp.int32)

    full = lambda shape: pl.BlockSpec(shape, lambda s, et_ref: (0,) * len(shape))
    w_specs = [pl.BlockSpec((1, H, H),
                            lambda s, et_ref, j=j: (et_ref[S * s + j], 0, 0))
               for j in range(S)]
    grid_spec = pltpu.PrefetchScalarGridSpec(
        num_scalar_prefetch=1,
        grid=(S,),
        in_specs=[
            full((S, H)),
            full((H, H)),
            full((H, H)),
            *w_specs,
            full((H, H)),
            full((1, H)),
            full((H, H)),
            full((H, H)),
            full((1, H)),
        ],
        out_specs=pl.BlockSpec((S, 2 * H), lambda s, et_ref: (0, 0)),
        scratch_shapes=[
            pltpu.VMEM((E, H), jnp.float32),
            pltpu.VMEM((S, H), jnp.float32),
        ],
    )
    return pl.pallas_call(
        _body,
        grid_spec=grid_spec,
        out_shape=jax.ShapeDtypeStruct((S, 2 * H), jnp.float32),
    )(et, global_features, Wq, Wk, *([rgcn_weight] * S), rgcn_root,
      rgcn_bias.reshape(1, H), gcn_lin_rel, gcn_lin_root,
      gcn_bias.reshape(1, H))


# speedup vs baseline: 119.1995x; 1.0958x over previous
import jax
import jax.numpy as jnp
from jax.experimental import pallas as pl
from jax.experimental.pallas import tpu as pltpu

S = 8
H = 256
E = S * S
K = 64
STEPS = E // K


def _body(et_ref, gf_ref, wq_ref, wk_ref, *rest):
    w_refs = rest[:K]
    (root_ref, rb_ref, lrel_ref, lroot_ref, gb_ref,
     out_ref, wgf_ref, acc_ref) = rest[K:]
    s = pl.program_id(0)

    @pl.when(s == 0)
    def _init():
        gf = gf_ref[...]
        q = jnp.dot(gf, wq_ref[...], preferred_element_type=jnp.float32)
        k = jnp.dot(gf, wk_ref[...], preferred_element_type=jnp.float32)
        scores = jnp.dot(q, k.T, preferred_element_type=jnp.float32) * (1.0 / 16.0)
        scores = scores - jnp.max(scores, axis=-1, keepdims=True)
        ex = jnp.exp(scores)
        attn = ex / jnp.sum(ex, axis=-1, keepdims=True)
        msgs = attn[:, :, None] * gf[:, None, :]
        wgf_ref[...] = msgs.reshape(E, H)
        acc_ref[...] = jnp.dot(gf, root_ref[...],
                               preferred_element_type=jnp.float32) + rb_ref[...]

    rows = wgf_ref[pl.ds(K * s, K), :]
    msgs = [jnp.dot(rows[j:j + 1, :], w_refs[j][0],
                    preferred_element_type=jnp.float32) for j in range(K)]
    total = jnp.concatenate(msgs[:S], axis=0)
    for b in range(1, K // S):
        total = total + jnp.concatenate(msgs[S * b:S * (b + 1)], axis=0)
    acc_ref[...] += total

    @pl.when(s == STEPS - 1)
    def _finish():
        x1 = acc_ref[...]
        agg = jnp.broadcast_to(jnp.sum(x1, axis=0, keepdims=True), (S, H))
        out2 = (jnp.dot(agg, lrel_ref[...], preferred_element_type=jnp.float32)
                + jnp.dot(x1, lroot_ref[...], preferred_element_type=jnp.float32)
                + gb_ref[...])
        out_ref[:, :H] = out2
        out_ref[:, H:] = gf_ref[...]


def kernel(global_features, speaker, Wq, Wk, Wv, rgcn_weight, rgcn_root,
           rgcn_bias, gcn_lin_rel, gcn_lin_root, gcn_bias):
    del Wv
    spk = speaker.astype(jnp.int32)
    src = jnp.repeat(jnp.arange(S, dtype=jnp.int32), S)
    dst = jnp.tile(jnp.arange(S, dtype=jnp.int32), S)
    et = (spk[src] * S + spk[dst]) * 2 + (src >= dst).astype(jnp.int32)

    full = lambda shape: pl.BlockSpec(shape, lambda s, et_ref: (0,) * len(shape))
    w_specs = [pl.BlockSpec((1, H, H),
                            lambda s, et_ref, j=j: (et_ref[K * s + j], 0, 0))
               for j in range(K)]
    grid_spec = pltpu.PrefetchScalarGridSpec(
        num_scalar_prefetch=1,
        grid=(STEPS,),
        in_specs=[
            full((S, H)),
            full((H, H)),
            full((H, H)),
            *w_specs,
            full((H, H)),
            full((1, H)),
            full((H, H)),
            full((H, H)),
            full((1, H)),
        ],
        out_specs=pl.BlockSpec((S, 2 * H), lambda s, et_ref: (0, 0)),
        scratch_shapes=[
            pltpu.VMEM((E, H), jnp.float32),
            pltpu.VMEM((S, H), jnp.float32),
        ],
    )
    return pl.pallas_call(
        _body,
        grid_spec=grid_spec,
        out_shape=jax.ShapeDtypeStruct((S, 2 * H), jnp.float32),
    )(et, global_features, Wq, Wk, *([rgcn_weight] * K), rgcn_root,
      rgcn_bias.reshape(1, H), gcn_lin_rel, gcn_lin_root,
      gcn_bias.reshape(1, H))
